# Initial kernel scaffold; baseline (speedup 1.0000x reference)
import jax
import jax.numpy as jnp
from jax.experimental import pallas as pl


def kernel(node_index, edge_index, edge_type, node_frequency, node_emb, comp1, bases1, root1, bias1, comp2, bases2, root2, bias2):
    raise NotImplementedError("write your pallas kernel here")


# speedup vs baseline: 4.3709x; 4.3709x over previous
import functools

import jax
import jax.numpy as jnp
from jax import lax
from jax.experimental import pallas as pl
from jax.experimental.pallas import tpu as pltpu
from jax.experimental.pallas import tpu_sc as plsc

_N = 10000
_NP = 10240
_E = 320000
_R = 8
_D = 128
_NB = 34
_NR = _N * _R

_NC = 2
_NS = 16
_NW = _NC * _NS
_EPW = _E // _NW
_CH = 80
_NCHUNK = _EPW // _CH
_RPS = _NP // _NS
_ST = 128
_CPS = _NR // _NS

_sc_mesh = plsc.VectorSubcoreMesh(core_axis_name="c", subcore_axis_name="s")


@functools.partial(
    pl.kernel,
    out_type=jax.ShapeDtypeStruct((_NC * _NR,), jnp.int32),
    mesh=_sc_mesh,
    scratch_types=[
        pltpu.VMEM_SHARED((_NR,), jnp.int32),
        pltpu.VMEM((_CH,), jnp.int32),
        pltpu.VMEM((_CH,), jnp.int32),
        pltpu.VMEM((_CH,), jnp.int32),
        pltpu.VMEM((_CH,), jnp.int32),
        pltpu.VMEM((_CPS,), jnp.int32),
    ],
)
def _sc_count(dst_hbm, rel_hbm, zero_hbm, ones_hbm, cnt_out,
              cnt_sp, dstb, relb, nkb, onesb, cstage):
    c = lax.axis_index("c")
    s = lax.axis_index("s")
    wid = s * _NC + c
    pltpu.sync_copy(zero_hbm.at[pl.ds(s * _CPS, _CPS)], cstage)
    pltpu.sync_copy(cstage, cnt_sp.at[pl.ds(s * _CPS, _CPS)])
    pltpu.sync_copy(ones_hbm, onesb)
    plsc.subcore_barrier()
    base0 = wid * _EPW

    @pl.loop(0, _NCHUNK)
    def _chunk(k):
        base = base0 + k * _CH
        pltpu.sync_copy(dst_hbm.at[pl.ds(base, _CH)], dstb)
        pltpu.sync_copy(rel_hbm.at[pl.ds(base, _CH)], relb)
        for j in range(_CH // 16):
            sl = pl.ds(j * 16, 16)
            nkb[sl] = dstb[sl] * _R + relb[sl]
        pltpu.sync_copy(onesb, cnt_sp.at[nkb], add=True)

    plsc.subcore_barrier()
    pltpu.sync_copy(cnt_sp.at[pl.ds(s * _CPS, _CPS)], cstage)
    pltpu.sync_copy(cstage, cnt_out.at[pl.ds(c * _NR + s * _CPS, _CPS)])


@functools.partial(
    pl.kernel,
    out_type=jax.ShapeDtypeStruct((_NC, _NP, _D), jnp.float32),
    mesh=_sc_mesh,
    scratch_types=[
        pltpu.VMEM_SHARED((_NP, _D), jnp.float32),
        pltpu.VMEM((_CH,), jnp.int32),
        pltpu.VMEM((_CH,), jnp.int32),
        pltpu.VMEM((_CH,), jnp.int32),
        pltpu.VMEM((_CH,), jnp.int32),
        pltpu.VMEM((_CH,), jnp.int32),
        pltpu.VMEM((_CH, _D), jnp.float32),
        pltpu.VMEM((_CH,), jnp.float32),
        pltpu.VMEM((_ST, _D), jnp.float32),
        pltpu.SemaphoreType.DMA,
        pltpu.SemaphoreType.DMA,
    ],
)
def _sc_layer(src_hbm, dst_hbm, rel_hbm, xw_hbm, inv_hbm, zero_hbm, acc_out,
              acc_sp, srcb, dstb, relb, gb, nkb, rowsb, invb, stage,
              sem_r, sem_i):
    c = lax.axis_index("c")
    s = lax.axis_index("s")
    wid = s * _NC + c
    pltpu.sync_copy(zero_hbm.at[pl.ds(0, _ST)], stage)
    for t in range(_RPS // _ST):
        pltpu.sync_copy(stage, acc_sp.at[pl.ds(s * _RPS + t * _ST, _ST)])
    plsc.subcore_barrier()
    base0 = wid * _EPW

    @pl.loop(0, _NCHUNK)
    def _chunk(k):
        base = base0 + k * _CH
        pltpu.sync_copy(src_hbm.at[pl.ds(base, _CH)], srcb)
        pltpu.sync_copy(dst_hbm.at[pl.ds(base, _CH)], dstb)
        pltpu.sync_copy(rel_hbm.at[pl.ds(base, _CH)], relb)
        for j in range(_CH // 16):
            sl = pl.ds(j * 16, 16)
            gb[sl] = relb[sl] * _NP + srcb[sl]
            nkb[sl] = dstb[sl] * _R + relb[sl]
        cp_rows = pltpu.async_copy(xw_hbm.at[gb], rowsb, sem_r)
        cp_inv = pltpu.async_copy(inv_hbm.at[nkb], invb, sem_i)
        cp_inv.wait()
        cp_rows.wait()

        @pl.loop(0, _CH // 16)
        def _scale(i16):
            i0 = i16 * 16
            iv = invb[pl.ds(i0, 16)]
            for t in range(16):
                sv = jnp.full((16,), iv[t])
                for j in range(_D // 16):
                    sl = pl.ds(j * 16, 16)
                    rowsb[i0 + t, sl] = rowsb[i0 + t, sl] * sv

        pltpu.sync_copy(rowsb, acc_sp.at[dstb], add=True)

    plsc.subcore_barrier()
    for t in range(_RPS // _ST):
        pltpu.sync_copy(acc_sp.at[pl.ds(s * _RPS + t * _ST, _ST)], stage)
        pltpu.sync_copy(stage, acc_out.at[c, pl.ds(s * _RPS + t * _ST, _ST)])


def _wmix_body(comp_ref, bases_ref, w_ref):
    w_ref[...] = jnp.dot(comp_ref[...], bases_ref[...],
                         preferred_element_type=jnp.float32)


def _wmix(comp, bases2d):
    return pl.pallas_call(
        _wmix_body,
        out_shape=jax.ShapeDtypeStruct((_R, _D * _D), jnp.float32),
    )(comp, bases2d)


_BN = 640
_NBLK = _NP // _BN
_CNT_ROWS = _NR // _D


def _dense1_body(x_ref, w_ref, root_ref, bias_ref, cnt_ref,
                 xw_ref, rootx_ref, inv_ref):
    xb = x_ref[...]
    for r in range(_R):
        xw_ref[r] = jnp.dot(xb, w_ref[r], preferred_element_type=jnp.float32)
    rootx_ref[...] = (jnp.dot(xb, root_ref[...],
                              preferred_element_type=jnp.float32)
                      + bias_ref[...])

    @pl.when(pl.program_id(0) == 0)
    def _():
        ctot = cnt_ref[0] + cnt_ref[1]
        inv_ref[...] = 1.0 / jnp.maximum(ctot, 1).astype(jnp.float32)


def _dense1(x, w, root, bias, cnt):
    return pl.pallas_call(
        _dense1_body,
        grid=(_NBLK,),
        in_specs=[
            pl.BlockSpec((_BN, _D), lambda i: (i, 0)),
            pl.BlockSpec((_R, _D, _D), lambda i: (0, 0, 0)),
            pl.BlockSpec((_D, _D), lambda i: (0, 0)),
            pl.BlockSpec((1, _D), lambda i: (0, 0)),
            pl.BlockSpec((_NC, _CNT_ROWS, _D), lambda i: (0, 0, 0)),
        ],
        out_specs=[
            pl.BlockSpec((_R, _BN, _D), lambda i: (0, i, 0)),
            pl.BlockSpec((_BN, _D), lambda i: (i, 0)),
            pl.BlockSpec((_CNT_ROWS, _D), lambda i: (0, 0)),
        ],
        out_shape=[
            jax.ShapeDtypeStruct((_R, _NP, _D), jnp.float32),
            jax.ShapeDtypeStruct((_NP, _D), jnp.float32),
            jax.ShapeDtypeStruct((_CNT_ROWS, _D), jnp.float32),
        ],
    )(x, w, root, bias, cnt)


def _dense2_body(acc_ref, rootx1_ref, w_ref, root_ref, bias_ref,
                 xw_ref, rootx_ref):
    xb = acc_ref[0] + acc_ref[1] + rootx1_ref[...]
    for r in range(_R):
        xw_ref[r] = jnp.dot(xb, w_ref[r], preferred_element_type=jnp.float32)
    rootx_ref[...] = (jnp.dot(xb, root_ref[...],
                              preferred_element_type=jnp.float32)
                      + bias_ref[...])


def _dense2(acc, rootx1, w, root, bias):
    return pl.pallas_call(
        _dense2_body,
        grid=(_NBLK,),
        in_specs=[
            pl.BlockSpec((_NC, _BN, _D), lambda i: (0, i, 0)),
            pl.BlockSpec((_BN, _D), lambda i: (i, 0)),
            pl.BlockSpec((_R, _D, _D), lambda i: (0, 0, 0)),
            pl.BlockSpec((_D, _D), lambda i: (0, 0)),
            pl.BlockSpec((1, _D), lambda i: (0, 0)),
        ],
        out_specs=[
            pl.BlockSpec((_R, _BN, _D), lambda i: (0, i, 0)),
            pl.BlockSpec((_BN, _D), lambda i: (i, 0)),
        ],
        out_shape=[
            jax.ShapeDtypeStruct((_R, _NP, _D), jnp.float32),
            jax.ShapeDtypeStruct((_NP, _D), jnp.float32),
        ],
    )(acc, rootx1, w, root, bias)


def _final_body(acc_ref, rootx_ref, out_ref):
    out_ref[...] = acc_ref[0] + acc_ref[1] + rootx_ref[...]


def _final(acc, rootx):
    return pl.pallas_call(
        _final_body,
        grid=(_NBLK,),
        in_specs=[
            pl.BlockSpec((_NC, _BN, _D), lambda i: (0, i, 0)),
            pl.BlockSpec((_BN, _D), lambda i: (i, 0)),
        ],
        out_specs=pl.BlockSpec((_BN, _D), lambda i: (i, 0)),
        out_shape=jax.ShapeDtypeStruct((_NP, _D), jnp.float32),
    )(acc, rootx)


def kernel(node_index, edge_index, edge_type, node_frequency, node_emb,
           comp1, bases1, root1, bias1, comp2, bases2, root2, bias2):
    del node_frequency
    x = node_emb[node_index]
    x = jnp.pad(x, ((0, _NP - _N), (0, 0)))
    srcs = edge_index[0].astype(jnp.int32)
    dsts = edge_index[1].astype(jnp.int32)
    rels = edge_type.astype(jnp.int32)
    zero_f = jnp.zeros((_NP, _D), jnp.float32)
    zero_i = jnp.zeros((_NR,), jnp.int32)
    ones_i = jnp.ones((_CH,), jnp.int32)

    cnt = _sc_count(dsts, rels, zero_i, ones_i)
    w1 = _wmix(comp1, bases1.reshape(_NB, _D * _D)).reshape(_R, _D, _D)
    w2 = _wmix(comp2, bases2.reshape(_NB, _D * _D)).reshape(_R, _D, _D)

    xw1, rootx1, inv = _dense1(x, w1, root1, bias1.reshape(1, _D),
                               cnt.reshape(_NC, _CNT_ROWS, _D))
    inv_flat = inv.reshape(_NR)
    acc1 = _sc_layer(srcs, dsts, rels, xw1.reshape(_R * _NP, _D),
                     inv_flat, zero_f)
    xw2, rootx2 = _dense2(acc1, rootx1, w2, root2, bias2.reshape(1, _D))
    acc2 = _sc_layer(srcs, dsts, rels, xw2.reshape(_R * _NP, _D),
                     inv_flat, zero_f)
    return _final(acc2, rootx2)[:_N]
